# baseline (speedup 1.0000x reference)
import functools

import jax
import jax.numpy as jnp
from jax import lax
from jax.experimental import pallas as pl
from jax.experimental.pallas import tpu as pltpu
from jax.experimental.pallas import tpu_sc as plsc

N_IMAGES = 2048
RES = 128
MIN_PDF = 0.01
NUM_SAMPLES = 65536
L = 16
NW = 32
S_W = NUM_SAMPLES // NW
BATCH = 128
NBATCH = S_W // BATCH


def _cdfy_body(em_ref, tri_ref, out_ref):
    em = em_ref[...]
    s = jnp.sum(em + 1e-10, axis=2)
    c = lax.dot_general(s, tri_ref[...], (((1,), (0,)), ((), ())),
                        precision=lax.Precision.HIGHEST,
                        preferred_element_type=jnp.float32)
    pdf_img = c[:, RES - 1:RES]
    liny = (lax.broadcasted_iota(jnp.int32, (1, RES), 1).astype(jnp.float32)
            + 1.0) / RES
    out_ref[...] = (1.0 - MIN_PDF) * c / pdf_img + MIN_PDF * liny


def _compute_cdf_y(error_map):
    B = 256
    tri = jnp.triu(jnp.ones((RES, RES), jnp.float32))
    return pl.pallas_call(
        _cdfy_body,
        grid=(N_IMAGES // B,),
        in_specs=[pl.BlockSpec((B, RES, RES), lambda i: (i, 0, 0)),
                  pl.BlockSpec((RES, RES), lambda i: (0, 0))],
        out_specs=pl.BlockSpec((B, RES), lambda i: (i, 0)),
        out_shape=jax.ShapeDtypeStruct((N_IMAGES, RES), jnp.float32),
    )(error_map, tri)


_MESH = plsc.VectorSubcoreMesh(core_axis_name="c", subcore_axis_name="s")


@functools.partial(
    pl.kernel,
    mesh=_MESH,
    out_type=[jax.ShapeDtypeStruct((NUM_SAMPLES,), jnp.float32),
              jax.ShapeDtypeStruct((NUM_SAMPLES,), jnp.float32)],
    scratch_types=[
        pltpu.VMEM((NBATCH, BATCH), jnp.int32),
        pltpu.VMEM((NBATCH, BATCH), jnp.int32),
        pltpu.VMEM((S_W,), jnp.float32),
        pltpu.VMEM((S_W,), jnp.float32),
        pltpu.VMEM((BATCH, RES), jnp.float32),
        pltpu.VMEM((BATCH, RES), jnp.float32),
        pltpu.VMEM((BATCH, RES), jnp.float32),
        pltpu.VMEM((BATCH, RES), jnp.float32),
        pltpu.VMEM((S_W,), jnp.float32),
        pltpu.VMEM((S_W,), jnp.float32),
        pltpu.SemaphoreType.DMA,
        pltpu.SemaphoreType.DMA,
    ],
    compiler_params=pltpu.CompilerParams(needs_layout_passes=False),
)
def _sample_kernel(cdfy_hbm, emflat_hbm, u3_hbm, fi3_hbm,
                   yout_hbm, xout_hbm,
                   fi2_v, fx2_v, ux_v, uy_v, yrA, yrB, xrA, xrB,
                   yo_v, xo_v, semA, semB):
    wid = lax.axis_index("s") * 2 + lax.axis_index("c")
    base = wid * S_W
    pltpu.sync_copy(fi3_hbm.at[wid], fi2_v)
    pltpu.sync_copy(u3_hbm.at[0, wid], ux_v)
    pltpu.sync_copy(u3_hbm.at[1, wid], uy_v)

    def ysearch(b, rows):
        @plsc.parallel_loop(0, BATCH // L, unroll=4)
        def ygroup(g):
            s0 = b * BATCH + g * L
            y = jnp.clip(uy_v[pl.ds(s0, L)], 1e-6, 1.0 - 1e-6)
            rowid = g * L + lax.iota(jnp.int32, L)
            pos = jnp.zeros((L,), jnp.int32)
            for ofs in (64, 32, 16, 8, 4, 2, 1):
                mid = pos + (ofs - 1)
                v = plsc.load_gather(rows, [rowid, mid])
                pos = jnp.where(v < y, pos + ofs, pos)
            h = pos
            prevv = plsc.load_gather(rows, [rowid, jnp.maximum(h - 1, 0)])
            prev = jnp.where(h > 0, prevv, 0.0)
            nxt = plsc.load_gather(rows, [rowid, h])
            yo_v[pl.ds(s0, L)] = ((y - prev) / (nxt - prev)
                                  + h.astype(jnp.float32)) * (1.0 / RES)
            fr = fi2_v[b, pl.ds(g * L, L)]
            fx2_v[b, pl.ds(g * L, L)] = fr * RES + h

    def xprocess(b, rows):
        @plsc.parallel_loop(0, BATCH, unroll=4)
        def _row(r):
            off = jnp.float32(0.0)
            for c in range(RES // L):
                chunk = rows[r, pl.ds(c * L, L)] + 1e-10
                scn = jnp.cumsum(chunk) + off
                rows[r, pl.ds(c * L, L)] = scn
                off = jnp.max(scn)

        @plsc.parallel_loop(0, BATCH // L, unroll=4)
        def xgroup(g):
            s0 = b * BATCH + g * L
            x = jnp.clip(ux_v[pl.ds(s0, L)], 1e-6, 1.0 - 1e-6)
            rowid = g * L + lax.iota(jnp.int32, L)
            tot = plsc.load_gather(
                rows, [rowid, jnp.full((L,), RES - 1, jnp.int32)])
            tscale = tot * (1.0 / (1.0 - MIN_PDF))
            pos = jnp.zeros((L,), jnp.int32)
            for ofs in (64, 32, 16, 8, 4, 2, 1):
                mid = pos + (ofs - 1)
                v = plsc.load_gather(rows, [rowid, mid])
                lin = (mid.astype(jnp.float32) + 1.0) * (MIN_PDF / RES)
                pos = jnp.where(v < (x - lin) * tscale, pos + ofs, pos)
            w = pos
            scprev = plsc.load_gather(rows, [rowid, jnp.maximum(w - 1, 0)])
            scnext = plsc.load_gather(rows, [rowid, w])
            nrm = (1.0 - MIN_PDF) / tot
            wf = w.astype(jnp.float32)
            prev = jnp.where(w > 0,
                             scprev * nrm + wf * (MIN_PDF / RES), 0.0)
            nxt = scnext * nrm + (wf + 1.0) * (MIN_PDF / RES)
            xo_v[pl.ds(s0, L)] = ((x - prev) / (nxt - prev) + wf) * (1.0 / RES)

    def y_issue(b, buf, sem):
        pltpu.async_copy(cdfy_hbm.at[fi2_v.at[b]], buf, sem)

    def y_wait(b, buf, sem):
        pltpu.make_async_copy(cdfy_hbm.at[fi2_v.at[b]], buf, sem).wait()

    y_issue(0, yrA, semA)

    def ypair(k, carry):
        b0 = 2 * k
        b1 = b0 + 1
        y_issue(b1, yrB, semB)
        y_wait(b0, yrA, semA)
        ysearch(b0, yrA)

        @pl.when(b1 + 1 < NBATCH)
        def _():
            y_issue(b1 + 1, yrA, semA)
        y_wait(b1, yrB, semB)
        ysearch(b1, yrB)
        return carry
    lax.fori_loop(0, NBATCH // 2, ypair, 0)

    def x_issue(b, buf, sem):
        pltpu.async_copy(emflat_hbm.at[fx2_v.at[b]], buf, sem)

    def x_wait(b, buf, sem):
        pltpu.make_async_copy(emflat_hbm.at[fx2_v.at[b]], buf, sem).wait()

    x_issue(0, xrA, semA)

    def xpair(k, carry):
        b0 = 2 * k
        b1 = b0 + 1
        x_issue(b1, xrB, semB)
        x_wait(b0, xrA, semA)
        xprocess(b0, xrA)

        @pl.when(b1 + 1 < NBATCH)
        def _():
            x_issue(b1 + 1, xrA, semA)
        x_wait(b1, xrB, semB)
        xprocess(b1, xrB)
        return carry
    lax.fori_loop(0, NBATCH // 2, xpair, 0)

    pltpu.sync_copy(yo_v, yout_hbm.at[pl.ds(base, S_W)])
    pltpu.sync_copy(xo_v, xout_hbm.at[pl.ds(base, S_W)])


def kernel(error_map, u, frame_ind, num_samples):
    cdf_y = _compute_cdf_y(error_map)
    em_flat = error_map.reshape(N_IMAGES * RES, RES)
    u3 = u.reshape(2, NW, S_W)
    fi3 = frame_ind.reshape(NW, NBATCH, BATCH)
    yo, xo = _sample_kernel(cdf_y, em_flat, u3, fi3)
    return jnp.stack([yo, xo], axis=0)


# speedup vs baseline: 1.0074x; 1.0074x over previous
import functools

import jax
import jax.numpy as jnp
from jax import lax
from jax.experimental import pallas as pl
from jax.experimental.pallas import tpu as pltpu
from jax.experimental.pallas import tpu_sc as plsc

N_IMAGES = 2048
RES = 128
MIN_PDF = 0.01
NUM_SAMPLES = 65536
L = 16
NW = 32
S_W = NUM_SAMPLES // NW
BATCH = 128
NBATCH = S_W // BATCH


def _cdfy_body(em_ref, tri_ref, out_ref):
    em = em_ref[...]
    s = jnp.sum(em + 1e-10, axis=2)
    c = lax.dot_general(s, tri_ref[...], (((1,), (0,)), ((), ())),
                        precision=lax.Precision.HIGHEST,
                        preferred_element_type=jnp.float32)
    pdf_img = c[:, RES - 1:RES]
    liny = (lax.broadcasted_iota(jnp.int32, (1, RES), 1).astype(jnp.float32)
            + 1.0) / RES
    out_ref[...] = (1.0 - MIN_PDF) * c / pdf_img + MIN_PDF * liny


def _compute_cdf_y(error_map):
    B = 256
    tri = jnp.triu(jnp.ones((RES, RES), jnp.float32))
    return pl.pallas_call(
        _cdfy_body,
        grid=(N_IMAGES // B,),
        in_specs=[pl.BlockSpec((B, RES, RES), lambda i: (i, 0, 0)),
                  pl.BlockSpec((RES, RES), lambda i: (0, 0))],
        out_specs=pl.BlockSpec((B, RES), lambda i: (i, 0)),
        out_shape=jax.ShapeDtypeStruct((N_IMAGES, RES), jnp.float32),
    )(error_map, tri)


_MESH = plsc.VectorSubcoreMesh(core_axis_name="c", subcore_axis_name="s")


@functools.partial(
    pl.kernel,
    mesh=_MESH,
    out_type=[jax.ShapeDtypeStruct((NUM_SAMPLES,), jnp.float32),
              jax.ShapeDtypeStruct((NUM_SAMPLES,), jnp.float32)],
    scratch_types=[
        pltpu.VMEM((NBATCH, BATCH), jnp.int32),
        pltpu.VMEM((NBATCH, BATCH), jnp.int32),
        pltpu.VMEM((S_W,), jnp.float32),
        pltpu.VMEM((S_W,), jnp.float32),
        pltpu.VMEM((BATCH, RES), jnp.float32),
        pltpu.VMEM((BATCH, RES), jnp.float32),
        pltpu.VMEM((BATCH, RES), jnp.float32),
        pltpu.VMEM((BATCH, RES), jnp.float32),
        pltpu.VMEM((S_W,), jnp.float32),
        pltpu.VMEM((S_W,), jnp.float32),
        pltpu.SemaphoreType.DMA,
        pltpu.SemaphoreType.DMA,
    ],
    compiler_params=pltpu.CompilerParams(needs_layout_passes=False),
)
def _sample_kernel(cdfy_hbm, emflat_hbm, u3_hbm, fi3_hbm,
                   yout_hbm, xout_hbm,
                   fi2_v, fx2_v, ux_v, uy_v, yrA, yrB, xrA, xrB,
                   yo_v, xo_v, semA, semB):
    wid = lax.axis_index("s") * 2 + lax.axis_index("c")
    base = wid * S_W
    pltpu.sync_copy(fi3_hbm.at[wid], fi2_v)
    pltpu.sync_copy(u3_hbm.at[0, wid], ux_v)
    pltpu.sync_copy(u3_hbm.at[1, wid], uy_v)

    def ysearch(b, rows):
        @plsc.parallel_loop(0, BATCH // L, unroll=2)
        def ygroup(g):
            s0 = b * BATCH + g * L
            y = jnp.clip(uy_v[pl.ds(s0, L)], 1e-6, 1.0 - 1e-6)
            rowid = g * L + lax.iota(jnp.int32, L)
            pos = jnp.zeros((L,), jnp.int32)
            for ofs in (64, 32, 16, 8, 4, 2, 1):
                mid = pos + (ofs - 1)
                v = plsc.load_gather(rows, [rowid, mid])
                pos = jnp.where(v < y, pos + ofs, pos)
            h = pos
            prevv = plsc.load_gather(rows, [rowid, jnp.maximum(h - 1, 0)])
            prev = jnp.where(h > 0, prevv, 0.0)
            nxt = plsc.load_gather(rows, [rowid, h])
            yo_v[pl.ds(s0, L)] = ((y - prev) / (nxt - prev)
                                  + h.astype(jnp.float32)) * (1.0 / RES)
            fr = fi2_v[b, pl.ds(g * L, L)]
            fx2_v[b, pl.ds(g * L, L)] = fr * RES + h

    def xprocess(b, rows):
        @plsc.parallel_loop(0, BATCH, unroll=4)
        def _row(r):
            off = jnp.float32(0.0)
            for c in range(RES // L):
                chunk = rows[r, pl.ds(c * L, L)] + 1e-10
                scn = jnp.cumsum(chunk) + off
                rows[r, pl.ds(c * L, L)] = scn
                off = jnp.max(scn)

        @plsc.parallel_loop(0, BATCH // L, unroll=2)
        def xgroup(g):
            s0 = b * BATCH + g * L
            x = jnp.clip(ux_v[pl.ds(s0, L)], 1e-6, 1.0 - 1e-6)
            rowid = g * L + lax.iota(jnp.int32, L)
            tot = plsc.load_gather(
                rows, [rowid, jnp.full((L,), RES - 1, jnp.int32)])
            tscale = tot * (1.0 / (1.0 - MIN_PDF))
            pos = jnp.zeros((L,), jnp.int32)
            for ofs in (64, 32, 16, 8, 4, 2, 1):
                mid = pos + (ofs - 1)
                v = plsc.load_gather(rows, [rowid, mid])
                lin = (mid.astype(jnp.float32) + 1.0) * (MIN_PDF / RES)
                pos = jnp.where(v < (x - lin) * tscale, pos + ofs, pos)
            w = pos
            scprev = plsc.load_gather(rows, [rowid, jnp.maximum(w - 1, 0)])
            scnext = plsc.load_gather(rows, [rowid, w])
            nrm = (1.0 - MIN_PDF) / tot
            wf = w.astype(jnp.float32)
            prev = jnp.where(w > 0,
                             scprev * nrm + wf * (MIN_PDF / RES), 0.0)
            nxt = scnext * nrm + (wf + 1.0) * (MIN_PDF / RES)
            xo_v[pl.ds(s0, L)] = ((x - prev) / (nxt - prev) + wf) * (1.0 / RES)

    def y_issue(b, buf, sem):
        pltpu.async_copy(cdfy_hbm.at[fi2_v.at[b]], buf, sem)

    def y_wait(b, buf, sem):
        pltpu.make_async_copy(cdfy_hbm.at[fi2_v.at[b]], buf, sem).wait()

    y_issue(0, yrA, semA)

    def ypair(k, carry):
        b0 = 2 * k
        b1 = b0 + 1
        y_issue(b1, yrB, semB)
        y_wait(b0, yrA, semA)
        ysearch(b0, yrA)

        @pl.when(b1 + 1 < NBATCH)
        def _():
            y_issue(b1 + 1, yrA, semA)
        y_wait(b1, yrB, semB)
        ysearch(b1, yrB)
        return carry
    lax.fori_loop(0, NBATCH // 2, ypair, 0)

    def x_issue(b, buf, sem):
        pltpu.async_copy(emflat_hbm.at[fx2_v.at[b]], buf, sem)

    def x_wait(b, buf, sem):
        pltpu.make_async_copy(emflat_hbm.at[fx2_v.at[b]], buf, sem).wait()

    x_issue(0, xrA, semA)

    def xpair(k, carry):
        b0 = 2 * k
        b1 = b0 + 1
        x_issue(b1, xrB, semB)
        x_wait(b0, xrA, semA)
        xprocess(b0, xrA)

        @pl.when(b1 + 1 < NBATCH)
        def _():
            x_issue(b1 + 1, xrA, semA)
        x_wait(b1, xrB, semB)
        xprocess(b1, xrB)
        return carry
    lax.fori_loop(0, NBATCH // 2, xpair, 0)

    pltpu.sync_copy(yo_v, yout_hbm.at[pl.ds(base, S_W)])
    pltpu.sync_copy(xo_v, xout_hbm.at[pl.ds(base, S_W)])


def kernel(error_map, u, frame_ind, num_samples):
    cdf_y = _compute_cdf_y(error_map)
    em_flat = error_map.reshape(N_IMAGES * RES, RES)
    u3 = u.reshape(2, NW, S_W)
    fi3 = frame_ind.reshape(NW, NBATCH, BATCH)
    yo, xo = _sample_kernel(cdf_y, em_flat, u3, fi3)
    return jnp.stack([yo, xo], axis=0)
